# Initial kernel scaffold; baseline (speedup 1.0000x reference)
import jax
import jax.numpy as jnp
from jax.experimental import pallas as pl


def kernel(h, W):
    raise NotImplementedError("write your pallas kernel here")


# speedup vs baseline: 1.1893x; 1.1893x over previous
import jax
import jax.numpy as jnp
from jax.experimental import pallas as pl
from jax.experimental.pallas import tpu as pltpu

N, P, D = 10000, 8, 256
BN = 400
NB = N // BN


def _pass1_body(h_ref, w_ref, beta_ref, acc_ref):
    i = pl.program_id(0)

    @pl.when(i == 0)
    def _init():
        acc_ref[...] = jnp.zeros_like(acc_ref)

    acc_ref[...] += jnp.sum(h_ref[...], axis=0)

    @pl.when(i == NB - 1)
    def _fin():
        w = jnp.dot(acc_ref[...], w_ref[...])
        w = jnp.where(w >= 0, w, 0.01 * w)
        m = jnp.max(w, axis=0, keepdims=True)
        e = jnp.exp(w - m)
        beta = e / jnp.sum(e, axis=0, keepdims=True)
        beta_ref[...] = jnp.broadcast_to(beta, (P, D))


def _pass2_body(h_ref, beta_ref, z_ref):
    z_ref[...] = jnp.sum(h_ref[...] * beta_ref[...][None, :, :], axis=1)


def kernel(h, W):
    beta_b = pl.pallas_call(
        _pass1_body,
        grid=(NB,),
        in_specs=[
            pl.BlockSpec((BN, P, D), lambda i: (i, 0, 0)),
            pl.BlockSpec((D, 1), lambda i: (0, 0)),
        ],
        out_specs=pl.BlockSpec((P, D), lambda i: (0, 0)),
        out_shape=jax.ShapeDtypeStruct((P, D), jnp.float32),
        scratch_shapes=[pltpu.VMEM((P, D), jnp.float32)],
    )(h, W)

    Z = pl.pallas_call(
        _pass2_body,
        grid=(NB,),
        in_specs=[
            pl.BlockSpec((BN, P, D), lambda i: (i, 0, 0)),
            pl.BlockSpec((P, D), lambda i: (0, 0)),
        ],
        out_specs=pl.BlockSpec((BN, D), lambda i: (i, 0)),
        out_shape=jax.ShapeDtypeStruct((N, D), jnp.float32),
    )(h, beta_b)
    return Z


# speedup vs baseline: 1.3990x; 1.1763x over previous
import jax
import jax.numpy as jnp
from jax.experimental import pallas as pl
from jax.experimental.pallas import tpu as pltpu

N, P, D = 10000, 8, 256
BN = 400
NB = N // BN
NRES = 14
NOT_RES = NB - NRES


def _order(k):
    return jnp.where(k < NOT_RES, NRES + k, k - NOT_RES)


def _body(h_ref, w_ref, z_ref, acc_ref, beta_ref, hres_ref):
    i = pl.program_id(0)

    @pl.when(i == 0)
    def _init():
        acc_ref[...] = jnp.zeros_like(acc_ref)

    @pl.when(i < NB)
    def _phase1():
        acc_ref[...] += jnp.sum(h_ref[...], axis=0)

        @pl.when(i < NRES)
        def _save():
            hres_ref[pl.ds(i * BN, BN)] = h_ref[...]

        @pl.when(i == NB - 1)
        def _beta():
            w = jnp.dot(acc_ref[...], w_ref[...])
            w = jnp.where(w >= 0, w, 0.01 * w)
            m = jnp.max(w, axis=0, keepdims=True)
            e = jnp.exp(w - m)
            beta = e / jnp.sum(e, axis=0, keepdims=True)
            beta_ref[...] = jnp.broadcast_to(beta, (P, D))

    @pl.when(i >= NB)
    def _phase2():
        k = i - NB
        b = beta_ref[...][None, :, :]

        @pl.when(k < NOT_RES)
        def _from_hbm():
            z_ref[...] = jnp.sum(h_ref[...] * b, axis=1)

        @pl.when(k >= NOT_RES)
        def _from_vmem():
            j = _order(k)
            z_ref[...] = jnp.sum(hres_ref[pl.ds(j * BN, BN)] * b, axis=1)


def _h_map(i):
    k = i - NB
    return (jnp.where(i < NB, i, jnp.minimum(NRES + k, NB - 1)), 0, 0)


def _z_map(i):
    return (_order(jnp.maximum(i - NB, 0)), 0)


def kernel(h, W):
    return pl.pallas_call(
        _body,
        grid=(2 * NB,),
        in_specs=[
            pl.BlockSpec((BN, P, D), _h_map),
            pl.BlockSpec((D, 1), lambda i: (0, 0)),
        ],
        out_specs=pl.BlockSpec((BN, D), _z_map),
        out_shape=jax.ShapeDtypeStruct((N, D), jnp.float32),
        scratch_shapes=[
            pltpu.VMEM((P, D), jnp.float32),
            pltpu.VMEM((P, D), jnp.float32),
            pltpu.VMEM((NRES * BN, P, D), jnp.float32),
        ],
    )(h, W)
